# baseline (speedup 1.0000x reference)
import jax
import jax.numpy as jnp
from jax.experimental import pallas as pl
from jax.experimental.pallas import tpu as pltpu

_N_BLK = 65536


def _pack_weights(w0, b0, w1, b1, w2, b2, w3, b3, w4, b4, w5, b5, w6, b6):
    return (jnp.zeros((64, 384), jnp.float32)
            .at[0:32, 0:4].set(w0.T).at[32:64, 4:8].set(w2.T)
            .at[0:32, 8].set(b0).at[32:64, 8].set(b2)
            .at[0:32, 9].set(b1).at[32:64, 9].set(b3)
            .at[0:32, 10].set(b4)
            .at[0:32, 11].set(b5)
            .at[0:8, 12].set(b6)
            .at[0:32, 128:160].set(w1.T).at[32:64, 160:192].set(w3.T)
            .at[0:32, 256:320].set(w4.T)
            .at[0:32, 320:352].set(w5.T)
            .at[0:8, 352:384].set(w6.T))


def _mlp_kernel(x_ref, s_ref, out_ref):
    bf16 = jnp.bfloat16
    s = s_ref[...]
    h = x_ref[...].astype(bf16)
    for a, c in (
        (s[:, 0:8], s[:, 8:9]),
        (s[:, 128:192], s[:, 9:10]),
        (s[0:32, 256:320], s[0:32, 10:11]),
        (s[0:32, 320:352], s[0:32, 11:12]),
    ):
        z = jnp.dot(a.astype(bf16), h, preferred_element_type=jnp.float32)
        h = jnp.maximum((z + c).astype(bf16), 0)
    out_ref[...] = (
        jnp.dot(s[0:8, 352:384].astype(bf16), h,
                preferred_element_type=jnp.float32)
        + s[0:8, 12:13])


def kernel(x, w0, b0, w1, b1, w2, b2, w3, b3, w4, b4, w5, b5, w6, b6):
    B, D = x.shape
    slab = _pack_weights(w0, b0, w1, b1, w2, b2, w3, b3, w4, b4, w5, b5,
                         w6, b6)

    xt = x.T
    b_pad = ((B + _N_BLK - 1) // _N_BLK) * _N_BLK
    if b_pad != B:
        xt = jnp.zeros((D, b_pad), xt.dtype).at[:, :B].set(xt)

    grid = (b_pad // _N_BLK,)
    cost = pl.CostEstimate(
        flops=2 * 8000 * b_pad,
        transcendentals=0,
        bytes_accessed=4 * 16 * b_pad,
    )
    out = pl.pallas_call(
        _mlp_kernel,
        out_shape=jax.ShapeDtypeStruct((8, b_pad), jnp.float32),
        grid=grid,
        in_specs=[
            pl.BlockSpec((8, _N_BLK), lambda i: (0, i)),
            pl.BlockSpec((64, 384), lambda i: (0, 0)),
        ],
        out_specs=pl.BlockSpec((8, _N_BLK), lambda i: (0, i)),
        compiler_params=pltpu.CompilerParams(
            dimension_semantics=("parallel",),
            allow_input_fusion=[False, False],
        ),
        cost_estimate=cost,
    )(xt, slab)

    return out[:, :B].T


# speedup vs baseline: 1.4099x; 1.4099x over previous
import jax
import jax.numpy as jnp
from jax.experimental import pallas as pl
from jax.experimental.pallas import tpu as pltpu

_N_BLK = 65536


def _pack_weights(w0, b0, w1, b1, w2, b2, w3, b3, w4, b4, w5, b5, w6, b6):
    f32 = jnp.float32
    bf16 = jnp.bfloat16

    a0 = (jnp.zeros((64, 8), f32)
          .at[:32, :4].set(w0.T).at[32:, 4:].set(w2.T)).astype(bf16)
    a1 = (jnp.zeros((64, 64), f32)
          .at[:32, :32].set(w1.T).at[32:, 32:].set(w3.T)).astype(bf16)
    a2 = w4.T.astype(bf16)
    a3 = w5.T.astype(bf16)
    a4 = w6.T.astype(bf16)
    c0 = jnp.concatenate([b0, b2])[:, None]
    c1 = jnp.concatenate([b1, b3])[:, None]
    return a0, a1, a2, a3, a4, c0, c1, b4[:, None], b5[:, None], b6[:, None]


def _mlp_kernel(x_ref, a0_ref, a1_ref, a2_ref, a3_ref, a4_ref,
                c0_ref, c1_ref, c2_ref, c3_ref, c4_ref, out_ref):
    bf16 = jnp.bfloat16
    h = x_ref[...].astype(bf16)
    for a_ref, c_ref in ((a0_ref, c0_ref), (a1_ref, c1_ref),
                         (a2_ref, c2_ref), (a3_ref, c3_ref)):
        z = jnp.dot(a_ref[...], h, preferred_element_type=jnp.float32)
        h = jnp.maximum((z + c_ref[...]).astype(bf16), 0)
    out_ref[...] = (
        jnp.dot(a4_ref[...], h, preferred_element_type=jnp.float32)
        + c4_ref[...])


def kernel(x, w0, b0, w1, b1, w2, b2, w3, b3, w4, b4, w5, b5, w6, b6):
    B, D = x.shape
    packed = _pack_weights(w0, b0, w1, b1, w2, b2, w3, b3, w4, b4, w5, b5,
                           w6, b6)

    xt = x.T
    b_pad = ((B + _N_BLK - 1) // _N_BLK) * _N_BLK
    if b_pad != B:
        xt = jnp.zeros((D, b_pad), xt.dtype).at[:, :B].set(xt)

    grid = (b_pad // _N_BLK,)
    cost = pl.CostEstimate(
        flops=2 * 8000 * b_pad,
        transcendentals=0,
        bytes_accessed=4 * 16 * b_pad,
    )
    wspecs = [pl.BlockSpec(w.shape, lambda i: (0, 0)) for w in packed]
    out = pl.pallas_call(
        _mlp_kernel,
        out_shape=jax.ShapeDtypeStruct((8, b_pad), jnp.float32),
        grid=grid,
        in_specs=[pl.BlockSpec((8, _N_BLK), lambda i: (0, i))] + wspecs,
        out_specs=pl.BlockSpec((8, _N_BLK), lambda i: (0, i)),
        compiler_params=pltpu.CompilerParams(
            dimension_semantics=("parallel",),
            allow_input_fusion=[False] + [True] * len(packed),
        ),
        cost_estimate=cost,
    )(xt, *packed)

    return out[:, :B].T
